# Initial kernel scaffold; baseline (speedup 1.0000x reference)
import jax
import jax.numpy as jnp
from jax.experimental import pallas as pl


def kernel(token_ids, table, fc_w, fc_b):
    raise NotImplementedError("write your pallas kernel here")


# speedup vs baseline: 3.0637x; 3.0637x over previous
import functools

import jax
import jax.numpy as jnp
from jax import lax
from jax.experimental import pallas as pl
from jax.experimental.pallas import tpu as pltpu
from jax.experimental.pallas import tpu_sc as plsc

NC = 2
NS = 16
NW = NC * NS
T_SPLIT = 2


def _sc_gather_sum(table, ids3, B, T, D, C):
    Tc = T // T_SPLIT
    RPW = B // NW
    n_chunks = RPW // C
    KD = D // 16

    mesh = plsc.VectorSubcoreMesh(core_axis_name="c", subcore_axis_name="s")

    @functools.partial(
        pl.kernel,
        out_type=jax.ShapeDtypeStruct((B, D), jnp.float32),
        mesh=mesh,
        compiler_params=pltpu.CompilerParams(use_tc_tiling_on_sc=False),
        scratch_types=[
            pltpu.VMEM((C, T_SPLIT, Tc), jnp.int32),
            pltpu.VMEM((2, T, D), jnp.float32),
            pltpu.VMEM((C, D), jnp.float32),
            pltpu.SemaphoreType.DMA,
            pltpu.SemaphoreType.DMA,
        ],
    )
    def k(table_hbm, ids_hbm, out_hbm, ids_v, rows_v, out_v, sem0, sem1):
        sems = (sem0, sem1)
        wid = lax.axis_index("s") * NC + lax.axis_index("c")
        base = wid * RPW

        def gather_copies(r, buf):
            return [
                pltpu.make_async_copy(
                    table_hbm.at[ids_v.at[r, j]],
                    rows_v.at[buf, pl.ds(j * Tc, Tc)],
                    sems[buf],
                )
                for j in range(T_SPLIT)
            ]

        def accumulate(buf):
            def body(t, carry):
                return tuple(
                    carry[kk] + rows_v[buf, t, pl.ds(kk * 16, 16)]
                    for kk in range(KD)
                )
            z = jnp.zeros((16,), jnp.float32)
            return lax.fori_loop(0, T, body, (z,) * KD, unroll=8)

        def chunk_body(ci, _):
            row0 = base + ci * C
            pltpu.sync_copy(ids_hbm.at[pl.ds(row0, C)], ids_v)
            for cp in gather_copies(0, 0):
                cp.start()
            for r in range(C):
                buf = r % 2
                if r + 1 < C:
                    for cp in gather_copies(r + 1, (r + 1) % 2):
                        cp.start()
                for cp in gather_copies(r, buf):
                    cp.wait()
                acc = accumulate(buf)
                for kk in range(KD):
                    out_v[r, pl.ds(kk * 16, 16)] = acc[kk]
            pltpu.sync_copy(out_v, out_hbm.at[pl.ds(row0, C)])
            return 0

        lax.fori_loop(0, n_chunks, chunk_body, 0)

    return k(table, ids3)


def _tc_epilogue(token_ids, sums, fc_w, fc_b2, B, T, D, NCLS, BB):
    def body(ids_ref, sums_ref, w_ref, b_ref, out_ref):
        ids = ids_ref[...]
        cnt = jnp.sum((ids != 0).astype(jnp.float32), axis=1, keepdims=True)
        pooled = sums_ref[...] / jnp.maximum(cnt, 1.0)
        logits = lax.dot_general(
            pooled, w_ref[...],
            dimension_numbers=(((1,), (1,)), ((), ())),
            preferred_element_type=jnp.float32,
        )
        out_ref[...] = logits + b_ref[...]

    return pl.pallas_call(
        body,
        grid=(B // BB,),
        in_specs=[
            pl.BlockSpec((BB, T), lambda i: (i, 0)),
            pl.BlockSpec((BB, D), lambda i: (i, 0)),
            pl.BlockSpec((NCLS, D), lambda i: (0, 0)),
            pl.BlockSpec((1, NCLS), lambda i: (0, 0)),
        ],
        out_specs=pl.BlockSpec((BB, NCLS), lambda i: (i, 0)),
        out_shape=jax.ShapeDtypeStruct((B, NCLS), jnp.float32),
    )(token_ids, sums, fc_w, fc_b2)


def kernel(token_ids, table, fc_w, fc_b):
    B, T = token_ids.shape
    V, D = table.shape
    NCLS = fc_w.shape[0]

    ids3 = token_ids.reshape(B, T_SPLIT, T // T_SPLIT)
    sums = _sc_gather_sum(table, ids3, B, T, D, C=16)
    return _tc_epilogue(token_ids, sums, fc_w, fc_b.reshape(1, NCLS),
                        B, T, D, NCLS, BB=2048)


# speedup vs baseline: 7.9580x; 2.5975x over previous
import functools

import jax
import jax.numpy as jnp
from jax import lax
from jax.experimental import pallas as pl
from jax.experimental.pallas import tpu as pltpu
from jax.experimental.pallas import tpu_sc as plsc

NC = 2
NS = 16
NW = NC * NS
_SLAB = 16384
_SQ = _SLAB // 4
_SQ_LOG2 = _SQ.bit_length() - 1
T_CHUNKS = (104, 96)


def _sigma_ids(ids):
    return (ids & -_SLAB) + ((ids & (_SQ - 1)) << 2) + ((ids >> _SQ_LOG2) & 3)


def _unpack_perm(D):
    perm = []
    for c in range(D // 32):
        perm += list(range(32 * c, 32 * c + 32, 2))
        perm += list(range(32 * c + 1, 32 * c + 32, 2))
    return perm


def _tc_relayout(table):
    V, D = table.shape
    SLAB = _SLAB
    nblk = pl.cdiv(V, SLAB)
    tbl_t = jnp.swapaxes(table, 0, 1)
    DW = D // 2
    q = SLAB // 4

    def body(in_ref, out_ref):
        ab = in_ref[...].astype(jnp.bfloat16)
        d_iota = jax.lax.broadcasted_iota(jnp.int32, (D, DW), 0)
        w_iota = jax.lax.broadcasted_iota(jnp.int32, (D, DW), 1)
        sel_e = (d_iota == 2 * w_iota).astype(jnp.bfloat16)
        sel_o = (d_iota == 2 * w_iota + 1).astype(jnp.bfloat16)
        ev = jax.lax.dot_general(
            ab, sel_e, (((0,), (0,)), ((), ())),
            preferred_element_type=jnp.float32)
        od = jax.lax.dot_general(
            ab, sel_o, (((0,), (0,)), ((), ())),
            preferred_element_type=jnp.float32)
        rnd = jnp.uint32(0x8000)
        evb = jax.lax.bitcast_convert_type(ev, jnp.uint32) + rnd
        odb = jax.lax.bitcast_convert_type(od, jnp.uint32) + rnd
        z = jax.lax.bitcast_convert_type(
            (evb >> 16) | (odb & jnp.uint32(0xFFFF0000)), jnp.int32)
        out_ref[...] = jnp.concatenate(
            [z[0:q], z[q:2 * q], z[2 * q:3 * q], z[3 * q:4 * q]], axis=1)

    out = pl.pallas_call(
        body,
        grid=(nblk,),
        in_specs=[pl.BlockSpec((D, SLAB), lambda i: (0, i))],
        out_specs=pl.BlockSpec((q, 128), lambda i: (i, 0)),
        out_shape=jax.ShapeDtypeStruct((nblk * q, 128), jnp.int32),
    )(tbl_t)
    return out.reshape(nblk * SLAB, DW)


def _sc_gather_sum(table, ids, B, T, D, C):
    RPW = B // NW
    n_chunks = RPW // C
    S0, S1 = T_CHUNKS

    mesh = plsc.VectorSubcoreMesh(core_axis_name="c", subcore_axis_name="s")

    @functools.partial(
        pl.kernel,
        out_type=jax.ShapeDtypeStruct((B, D), jnp.float32),
        mesh=mesh,
        compiler_params=pltpu.CompilerParams(
            use_tc_tiling_on_sc=False, needs_layout_passes=False),
        scratch_types=[
            pltpu.VMEM((2, C * T), jnp.int32),
            pltpu.VMEM((8, T, D // 2), jnp.int32),
            pltpu.VMEM((2, C, D), jnp.float32),
            pltpu.SemaphoreType.DMA,
            pltpu.SemaphoreType.DMA,
            pltpu.SemaphoreType.DMA,
            pltpu.SemaphoreType.DMA,
            pltpu.SemaphoreType.DMA,
            pltpu.SemaphoreType.DMA,
            pltpu.SemaphoreType.DMA,
            pltpu.SemaphoreType.DMA,
            pltpu.SemaphoreType.DMA,
            pltpu.SemaphoreType.DMA,
        ],
    )
    def k(table_hbm, ids_hbm, out_hbm, ids_v, rows_v, out_v,
          gsem0, gsem1, gsem2, gsem3, gsem4, gsem5, gsem6, gsem7,
          ids_sem, out_sem):
        gsems = (gsem0, gsem1, gsem2, gsem3, gsem4, gsem5, gsem6, gsem7)
        wid = lax.axis_index("s") * NC + lax.axis_index("c")
        base = wid * RPW

        def ids_copy(ci, ib):
            return pltpu.make_async_copy(
                ids_hbm.at[pl.ds((base + ci * C) * T, C * T)],
                ids_v.at[ib], ids_sem)

        def gather_copies(r, buf, ib):
            return [
                pltpu.make_async_copy(
                    table_hbm.at[ids_v.at[ib, pl.ds(r * T, S0)]],
                    rows_v.at[buf, pl.ds(0, S0)],
                    gsems[buf],
                ),
                pltpu.make_async_copy(
                    table_hbm.at[ids_v.at[ib, pl.ds(r * T + S0, S1)]],
                    rows_v.at[buf, pl.ds(S0, S1)],
                    gsems[buf],
                ),
            ]

        def out_copy(ci, ob):
            return pltpu.make_async_copy(
                out_v.at[ob], out_hbm.at[pl.ds(base + ci * C, C)], out_sem)

        def accumulate(buf):
            mask = jnp.full((16,), -65536, jnp.int32)

            def body(t, carry):
                acc = list(carry)
                for c in range(D // 32):
                    x = rows_v[buf, t, pl.ds(16 * c, 16)]
                    even = plsc.bitcast(x << 16, jnp.float32)
                    odd = plsc.bitcast(x & mask, jnp.float32)
                    acc[2 * c] = acc[2 * c] + even
                    acc[2 * c + 1] = acc[2 * c + 1] + odd
                return tuple(acc)
            z = jnp.zeros((16,), jnp.float32)
            return lax.fori_loop(0, T, body, (z,) * (D // 16), unroll=8)

        def process_chunk(ci2, par):
            ci = 2 * ci2 + par
            for r in range(C):
                buf = r % 8
                tgt = r + 6
                if tgt < C:
                    for cp in gather_copies(tgt, tgt % 8, par):
                        cp.start()
                else:
                    nt = tgt - C
                    @pl.when(ci + 1 < n_chunks)
                    def _():
                        if nt == 0:
                            ids_copy(ci + 1, 1 - par).wait()
                        for cp in gather_copies(nt, nt % 8, 1 - par):
                            cp.start()
                for cp in gather_copies(r, buf, par):
                    cp.wait()
                if r == C - 1:
                    @pl.when(ci + 2 < n_chunks)
                    def _():
                        ids_copy(ci + 2, par).start()
                acc = accumulate(buf)
                for kk in range(D // 16):
                    out_v[par, r, pl.ds(kk * 16, 16)] = acc[kk]
            if par == 0:
                @pl.when(ci2 > 0)
                def _():
                    out_copy(2 * ci2 - 1, 1).wait()
            else:
                out_copy(2 * ci2, 0).wait()
            out_copy(ci, par).start()

        def chunk_pair(ci2, _):
            process_chunk(ci2, 0)
            process_chunk(ci2, 1)
            return 0

        ids_copy(0, 0).start()
        ids_copy(0, 0).wait()
        ids_copy(1, 1).start()
        for r0 in range(6):
            for cp in gather_copies(r0, r0, 0):
                cp.start()
        lax.fori_loop(0, n_chunks // 2, chunk_pair, 0)
        out_copy(n_chunks - 1, 1).wait()

    return k(table, ids)


def _tc_epilogue(token_ids, sums, fc_w, fc_b2, B, T, D, NCLS, BB):
    ids_t = jnp.swapaxes(token_ids, 0, 1)

    def body(ids_ref, sums_ref, w_ref, b_ref, out_ref):
        nz = (ids_ref[...] != 0).astype(jnp.float32)
        cnt = jnp.sum(nz, axis=0)
        pooled = sums_ref[...] / jnp.maximum(cnt, 1.0)[:, None]
        logits = lax.dot_general(
            pooled, w_ref[...],
            dimension_numbers=(((1,), (1,)), ((), ())),
            preferred_element_type=jnp.float32,
        )
        out_ref[...] = logits + b_ref[...]

    return pl.pallas_call(
        body,
        grid=(B // BB,),
        in_specs=[
            pl.BlockSpec((T, BB), lambda i: (0, i)),
            pl.BlockSpec((BB, D), lambda i: (i, 0)),
            pl.BlockSpec((NCLS, D), lambda i: (0, 0)),
            pl.BlockSpec((1, NCLS), lambda i: (0, 0)),
        ],
        out_specs=pl.BlockSpec((BB, NCLS), lambda i: (i, 0)),
        out_shape=jax.ShapeDtypeStruct((B, NCLS), jnp.float32),
    )(ids_t, sums, fc_w, fc_b2)


def kernel(token_ids, table, fc_w, fc_b):
    B, T = token_ids.shape
    V, D = table.shape
    NCLS = fc_w.shape[0]

    ids1 = _sigma_ids(token_ids).reshape(B * T)
    tbl16 = _tc_relayout(table)
    sums = _sc_gather_sum(tbl16, ids1, B, T, D, C=16)
    fc_wp = fc_w[:, jnp.array(_unpack_perm(D), dtype=jnp.int32)]
    return _tc_epilogue(token_ids, sums, fc_wp, fc_b.reshape(1, NCLS),
                        B, T, D, NCLS, BB=2048)
